# Initial kernel scaffold; baseline (speedup 1.0000x reference)
import jax
import jax.numpy as jnp
from jax.experimental import pallas as pl


def kernel(x, opcode, pc, imm, ax):
    raise NotImplementedError("write your pallas kernel here")


# speedup vs baseline: 6.2248x; 6.2248x over previous
import jax
import jax.numpy as jnp
from jax.experimental import pallas as pl
from jax.experimental.pallas import tpu as pltpu

_DIM = 1280
_DIM_PER_POS = 160
_NUM_POS = 8
_ROWS = 4 * 8192
_BLOCK_ROWS = 1024


def _next_pc_scalar(opcode, pc, imm, ax):
    seq_pc = pc + 8
    return jnp.where(
        opcode == 1,
        imm,
        jnp.where(
            opcode == 2,
            jnp.where(ax == 0, imm, seq_pc),
            jnp.where(
                opcode == 3,
                jnp.where(ax != 0, imm, seq_pc),
                jnp.where(opcode == 4, imm, seq_pc),
            ),
        ),
    )


def _body(scalars_ref, x_ref, o_ref):
    opcode = scalars_ref[0]
    pc = scalars_ref[1]
    imm = scalars_ref[2]
    ax = scalars_ref[3]
    next_pc = _next_pc_scalar(opcode, pc, imm, ax)

    col = jax.lax.broadcasted_iota(jnp.int32, (1, _DIM), 1)
    pos = col // _DIM_PER_POS
    nib = jax.lax.shift_right_arithmetic(next_pc, pos * 4) & 15
    mask = (col % _DIM_PER_POS) == 0
    o_ref[...] = jnp.where(mask, nib.astype(jnp.float32), x_ref[...])


def kernel(x, opcode, pc, imm, ax):
    orig_shape = x.shape
    x2 = x.reshape(_ROWS, _DIM)
    scalars = jnp.array([opcode, pc, imm, ax], dtype=jnp.int32)
    out = pl.pallas_call(
        _body,
        grid=(_ROWS // _BLOCK_ROWS,),
        in_specs=[
            pl.BlockSpec(memory_space=pltpu.SMEM),
            pl.BlockSpec((_BLOCK_ROWS, _DIM), lambda i: (i, 0)),
        ],
        out_specs=pl.BlockSpec((_BLOCK_ROWS, _DIM), lambda i: (i, 0)),
        out_shape=jax.ShapeDtypeStruct((_ROWS, _DIM), jnp.float32),
    )(scalars, x2)
    return out.reshape(orig_shape)


# speedup vs baseline: 6.3038x; 1.0127x over previous
import jax
import jax.numpy as jnp
from jax.experimental import pallas as pl
from jax.experimental.pallas import tpu as pltpu

_DIM = 1280
_DIM_PER_POS = 160
_NUM_POS = 8
_ROWS = 4 * 8192
_BLOCK_ROWS = 2048


def _next_pc_scalar(opcode, pc, imm, ax):
    seq_pc = pc + 8
    return jnp.where(
        opcode == 1,
        imm,
        jnp.where(
            opcode == 2,
            jnp.where(ax == 0, imm, seq_pc),
            jnp.where(
                opcode == 3,
                jnp.where(ax != 0, imm, seq_pc),
                jnp.where(opcode == 4, imm, seq_pc),
            ),
        ),
    )


def _body(scalars_ref, x_ref, o_ref):
    opcode = scalars_ref[0]
    pc = scalars_ref[1]
    imm = scalars_ref[2]
    ax = scalars_ref[3]
    next_pc = _next_pc_scalar(opcode, pc, imm, ax)

    col = jax.lax.broadcasted_iota(jnp.int32, (1, _DIM), 1)
    pos = col // _DIM_PER_POS
    nib = jax.lax.shift_right_arithmetic(next_pc, pos * 4) & 15
    mask = (col % _DIM_PER_POS) == 0
    o_ref[...] = jnp.where(mask, nib.astype(jnp.float32), x_ref[...])


def kernel(x, opcode, pc, imm, ax):
    orig_shape = x.shape
    x2 = x.reshape(_ROWS, _DIM)
    scalars = jnp.array([opcode, pc, imm, ax], dtype=jnp.int32)
    out = pl.pallas_call(
        _body,
        grid=(_ROWS // _BLOCK_ROWS,),
        in_specs=[
            pl.BlockSpec(memory_space=pltpu.SMEM),
            pl.BlockSpec((_BLOCK_ROWS, _DIM), lambda i: (i, 0)),
        ],
        out_specs=pl.BlockSpec((_BLOCK_ROWS, _DIM), lambda i: (i, 0)),
        out_shape=jax.ShapeDtypeStruct((_ROWS, _DIM), jnp.float32),
    )(scalars, x2)
    return out.reshape(orig_shape)
